# Initial kernel scaffold; baseline (speedup 1.0000x reference)
import jax
import jax.numpy as jnp
from jax.experimental import pallas as pl


def kernel(x, edge_index, W1_0, W1_1, W1_2, b1, W2_0, W2_1, W2_2, b2, W3_0, W3_1, W3_2, b3):
    raise NotImplementedError("write your pallas kernel here")


# speedup vs baseline: 2.4266x; 2.4266x over previous
import functools

import jax
import jax.numpy as jnp
from jax import lax
from jax.experimental import pallas as pl
from jax.experimental.pallas import tpu as pltpu
from jax.experimental.pallas import tpu_sc as plsc

N = 10000
NP = 10240
E = 320000
EP = 327680
NW = 32
EPW = EP // NW
BB = 128
NB = EPW // BB
RB = 256
NM = NP // RB
STRIPE = NP // 16
C = 128


def _mesh():
    return plsc.VectorSubcoreMesh(
        core_axis_name="c", subcore_axis_name="s", num_cores=2, num_subcores=16
    )


_SC_PARAMS = pltpu.CompilerParams(needs_layout_passes=False)


def _sc_deg(src_flat, dst_flat):
    @functools.partial(
        pl.kernel,
        mesh=_mesh(),
        out_type=(
            jax.ShapeDtypeStruct((NW, NP), jnp.float32),
            jax.ShapeDtypeStruct((NW, EPW), jnp.int32),
        ),
        scratch_types=[
            pltpu.VMEM((EPW,), jnp.int32),
            pltpu.VMEM((EPW,), jnp.int32),
            pltpu.VMEM((EPW,), jnp.int32),
            pltpu.VMEM((NP,), jnp.float32),
        ],
        compiler_params=_SC_PARAMS,
    )
    def k(src_hbm, dst_hbm, deg_hbm, srcz_hbm, src_v, dst_v, srcz_v, hist_v):
        cc = lax.axis_index("c")
        ss = lax.axis_index("s")
        wid = cc * 16 + ss
        pltpu.sync_copy(src_hbm.at[wid], src_v)
        pltpu.sync_copy(dst_hbm.at[wid], dst_v)

        def zbody(i, carry):
            hist_v[pl.ds(i * 16, 16)] = jnp.zeros((16,), jnp.float32)
            return carry

        lax.fori_loop(0, NP // 16, zbody, 0)

        ones = jnp.ones((16,), jnp.float32)

        def ebody(i, carry):
            s16 = src_v[pl.ds(i * 16, 16)]
            d16 = dst_v[pl.ds(i * 16, 16)]
            m = s16 != d16
            plsc.addupdate_scatter(hist_v, [s16], ones, mask=m)
            srcz_v[pl.ds(i * 16, 16)] = jnp.where(m, s16, N)
            return carry

        lax.fori_loop(0, EPW // 16, ebody, 0)
        pltpu.sync_copy(hist_v, deg_hbm.at[wid])
        pltpu.sync_copy(srcz_v, srcz_hbm.at[wid])

    return k(src_flat, dst_flat)


def _sc_prop(g, srcz_r, dst_r, zero_stripe):
    nc = g.shape[0]

    @functools.partial(
        pl.kernel,
        mesh=_mesh(),
        out_type=jax.ShapeDtypeStruct((2, nc, NP, C), jnp.float32),
        scratch_types=[
            pltpu.VMEM((NB, BB), jnp.int32),
            pltpu.VMEM((NB, BB), jnp.int32),
            pltpu.VMEM((BB, C), jnp.float32),
            pltpu.VMEM_SHARED((NP, C), jnp.float32),
            pltpu.SemaphoreType.DMA,
        ],
        compiler_params=_SC_PARAMS,
    )
    def k(g_hbm, srcz_hbm, dst_hbm, zero_hbm, s_hbm, srcz_v, dst_v, rows_v, acc_sh, sem):
        cc = lax.axis_index("c")
        ss = lax.axis_index("s")
        wid = cc * 16 + ss
        pltpu.sync_copy(srcz_hbm.at[wid], srcz_v)
        pltpu.sync_copy(dst_hbm.at[wid], dst_v)
        for ci in range(nc):
            pltpu.sync_copy(zero_hbm, acc_sh.at[pl.ds(ss * STRIPE, STRIPE)])
            plsc.subcore_barrier()

            def body(b, carry):
                pltpu.async_copy(g_hbm.at[ci].at[srcz_v.at[b]], rows_v, sem).wait()
                pltpu.sync_copy(rows_v, acc_sh.at[dst_v.at[b]], add=True)
                return carry

            lax.fori_loop(0, NB, body, 0)
            plsc.subcore_barrier()
            pltpu.sync_copy(
                acc_sh.at[pl.ds(ss * STRIPE, STRIPE)],
                s_hbm.at[cc].at[ci].at[pl.ds(ss * STRIPE, STRIPE)],
            )

    return k(g, srcz_r, dst_r, zero_stripe)


def _dinv_of(deg_blk):
    deg = jnp.sum(deg_blk, axis=0)
    return jnp.where(deg > 0, lax.rsqrt(deg), 0.0)


def _tc_g0(x_c, deg_p):
    def body(x_ref, deg_ref, g_ref):
        dinv = _dinv_of(deg_ref[...])
        g_ref[0] = dinv[:, None] * x_ref[0]

    return pl.pallas_call(
        body,
        grid=(NM,),
        in_specs=[
            pl.BlockSpec((1, RB, C), lambda m: (0, m, 0)),
            pl.BlockSpec((NW, RB), lambda m: (0, m)),
        ],
        out_specs=pl.BlockSpec((1, RB, C), lambda m: (0, m, 0)),
        out_shape=jax.ShapeDtypeStruct((1, NP, C), jnp.float32),
    )(x_c, deg_p)


def _tc_mid(s1p, deg_p):
    nc = s1p.shape[1]

    def body(s_ref, deg_ref, t1_ref, g1_ref):
        dinv = _dinv_of(deg_ref[...])
        ssum = s_ref[0, 0] + s_ref[1, 0]
        t1 = -dinv[:, None] * ssum
        t1_ref[0] = t1
        g1_ref[0] = dinv[:, None] * t1

    return pl.pallas_call(
        body,
        grid=(nc, NM),
        in_specs=[
            pl.BlockSpec((2, 1, RB, C), lambda k, m: (0, k, m, 0)),
            pl.BlockSpec((NW, RB), lambda k, m: (0, m)),
        ],
        out_specs=[
            pl.BlockSpec((1, RB, C), lambda k, m: (k, m, 0)),
            pl.BlockSpec((1, RB, C), lambda k, m: (k, m, 0)),
        ],
        out_shape=[
            jax.ShapeDtypeStruct((nc, NP, C), jnp.float32),
            jax.ShapeDtypeStruct((nc, NP, C), jnp.float32),
        ],
    )(s1p, deg_p)


def _tc_out(h_c, t1_c, s2p, deg_p, Wcat, b2d, final):
    nc = h_c.shape[0]
    din = nc * C
    dout = Wcat.shape[1]
    mc = dout // C

    def body(h_ref, t1_ref, s_ref, deg_ref, w_ref, b_ref, *out_refs):
        dinv = _dinv_of(deg_ref[...])
        acc = jnp.zeros((RB, C), jnp.float32)
        for kk in range(nc):
            hk = h_ref[kk]
            uk = -2.0 * dinv[:, None] * (s_ref[0, kk] + s_ref[1, kk])
            w0 = w_ref[kk * C:(kk + 1) * C, :]
            w1 = w_ref[din + kk * C:din + (kk + 1) * C, :]
            w2 = w_ref[2 * din + kk * C:2 * din + (kk + 1) * C, :]
            acc = acc + jnp.dot(hk, w0, preferred_element_type=jnp.float32)
            acc = acc + jnp.dot(t1_ref[kk], w1, preferred_element_type=jnp.float32)
            acc = acc + jnp.dot(uk - hk, w2, preferred_element_type=jnp.float32)
        out = jnp.maximum(acc + b_ref[...], 0.0)
        if final:
            out_refs[0][...] = out
        else:
            out_refs[0][0] = out
            out_refs[1][0] = dinv[:, None] * out

    in_specs = [
        pl.BlockSpec((nc, RB, C), lambda m, n: (0, m, 0)),
        pl.BlockSpec((nc, RB, C), lambda m, n: (0, m, 0)),
        pl.BlockSpec((2, nc, RB, C), lambda m, n: (0, 0, m, 0)),
        pl.BlockSpec((NW, RB), lambda m, n: (0, m)),
        pl.BlockSpec((3 * din, C), lambda m, n: (0, n)),
        pl.BlockSpec((1, C), lambda m, n: (0, n)),
    ]
    if final:
        out_specs = pl.BlockSpec((RB, C), lambda m, n: (m, n))
        out_shape = jax.ShapeDtypeStruct((NP, dout), jnp.float32)
    else:
        out_specs = [
            pl.BlockSpec((1, RB, C), lambda m, n: (n, m, 0)),
            pl.BlockSpec((1, RB, C), lambda m, n: (n, m, 0)),
        ]
        out_shape = [
            jax.ShapeDtypeStruct((mc, NP, C), jnp.float32),
            jax.ShapeDtypeStruct((mc, NP, C), jnp.float32),
        ]
    return pl.pallas_call(
        body,
        grid=(NM, mc),
        in_specs=in_specs,
        out_specs=out_specs,
        out_shape=out_shape,
    )(h_c, t1_c, s2p, deg_p, Wcat, b2d)


def kernel(x, edge_index, W1_0, W1_1, W1_2, b1, W2_0, W2_1, W2_2, b2,
           W3_0, W3_1, W3_2, b3):
    src = jnp.pad(edge_index[0], (0, EP - E)).reshape(NW, EPW)
    dst = jnp.pad(edge_index[1], (0, EP - E)).reshape(NW, EPW)
    x_c = jnp.pad(x, ((0, NP - N), (0, 0))).reshape(1, NP, C)
    zero_stripe = jnp.zeros((STRIPE, C), jnp.float32)

    deg_p, srcz = _sc_deg(src, dst)
    srcz_r = srcz.reshape(NW, NB, BB)
    dst_r = dst.reshape(NW, NB, BB)

    h = x_c
    g = _tc_g0(x_c, deg_p)
    layers = [
        (W1_0, W1_1, W1_2, b1),
        (W2_0, W2_1, W2_2, b2),
        (W3_0, W3_1, W3_2, b3),
    ]
    for li, (W0, W1, W2, b) in enumerate(layers):
        final = li == len(layers) - 1
        s1p = _sc_prop(g, srcz_r, dst_r, zero_stripe)
        t1, g1 = _tc_mid(s1p, deg_p)
        s2p = _sc_prop(g1, srcz_r, dst_r, zero_stripe)
        Wcat = jnp.concatenate([W0, W1, W2], axis=0)
        b2d = b.reshape(1, -1)
        if final:
            h = _tc_out(h, t1, s2p, deg_p, Wcat, b2d, final=True)
        else:
            h, g = _tc_out(h, t1, s2p, deg_p, Wcat, b2d, final=False)
    return h[:N]
